# baseline (speedup 1.0000x reference)
import functools

import jax
import jax.numpy as jnp
from jax import lax
from jax.experimental import pallas as pl
from jax.experimental.pallas import tpu as pltpu
from jax.experimental.pallas import tpu_sc as plsc

_TEMPERATURE = 0.1
_B = 128
_B2 = 2 * _B
_D = 64
_Q = 1000000
_BQ = 10000
_NBLK = _Q // _BQ
_NEG = -3.0e38
_BIGF = 3.0e38


def _simarg_body(p_ref, q_ref, idx_out, pn_out, vmax, vidx, viota):
    i = pl.program_id(0)

    @pl.when(i == 0)
    def _init():
        vmax[...] = jnp.full((1, _B2), _NEG, jnp.float32)
        vidx[...] = jnp.zeros((1, _B2), jnp.int32)
        viota[...] = lax.broadcasted_iota(
            jnp.int32, (_BQ, _B2), 0
        ).astype(jnp.float32)

    p = p_ref[...]
    sq = jnp.sum(p * p, axis=1, keepdims=True)
    pn = p * lax.rsqrt(jnp.maximum(sq, 1e-12))

    q = q_ref[...]
    sim = lax.dot_general(
        q, pn, (((1,), (1,)), ((), ())),
        preferred_element_type=jnp.float32,
    )
    bmax = jnp.max(sim, axis=0, keepdims=True)
    bloc = jnp.min(
        jnp.where(sim == bmax, viota[...], _BIGF), axis=0, keepdims=True
    )
    bidx = bloc.astype(jnp.int32) + i * _BQ
    better = bmax > vmax[...]
    vmax[...] = jnp.where(better, bmax, vmax[...])
    vidx[...] = jnp.where(better, bidx, vidx[...])

    @pl.when(i == _NBLK - 1)
    def _fin():
        idx_out[...] = vidx[...]
        pn_out[...] = pn


def _simarg(P, Qf):
    return pl.pallas_call(
        _simarg_body,
        grid=(_NBLK,),
        in_specs=[
            pl.BlockSpec((_B2, _D), lambda i: (0, 0)),
            pl.BlockSpec((_BQ, _D), lambda i: (i, 0)),
        ],
        out_specs=[
            pl.BlockSpec((1, _B2), lambda i: (0, 0)),
            pl.BlockSpec((_B2, _D), lambda i: (0, 0)),
        ],
        out_shape=[
            jax.ShapeDtypeStruct((1, _B2), jnp.int32),
            jax.ShapeDtypeStruct((_B2, _D), jnp.float32),
        ],
        scratch_shapes=[
            pltpu.VMEM((1, _B2), jnp.float32),
            pltpu.VMEM((1, _B2), jnp.int32),
            pltpu.VMEM((_BQ, _B2), jnp.float32),
        ],
    )(P, Qf)


def _sc_gather(Q, idx):
    info = plsc.get_sparse_core_info()
    nw = info.num_cores * info.num_subcores
    bpw = _B2 // nw

    mesh = plsc.VectorSubcoreMesh(core_axis_name="c", subcore_axis_name="s")

    @functools.partial(
        pl.kernel,
        mesh=mesh,
        compiler_params=pltpu.CompilerParams(use_tc_tiling_on_sc=False),
        out_type=jax.ShapeDtypeStruct((_B2, _D), jnp.float32),
        scratch_types=[
            pltpu.VMEM((bpw,), jnp.int32),
            pltpu.VMEM((bpw, _D), jnp.float32),
            pltpu.SemaphoreType.DMA,
        ],
    )
    def gk(q_hbm, idx_hbm, out_hbm, idx_v, rows_v, sem):
        wid = lax.axis_index("s") * info.num_cores + lax.axis_index("c")
        base = wid * bpw
        pltpu.sync_copy(idx_hbm.at[pl.ds(base, bpw)], idx_v)
        pltpu.async_copy(q_hbm.at[idx_v], rows_v, sem).wait()
        pltpu.sync_copy(rows_v, out_hbm.at[pl.ds(base, bpw)])

    return gk(Q, idx)


def _loss_body(pn_ref, nn_ref, out_ref):
    pn = pn_ref[...]
    nn = nn_ref[...]
    p1 = pn[:_B]
    p2 = pn[_B:]
    n1 = nn[:_B]
    n2 = nn[_B:]
    inv_t = 1.0 / _TEMPERATURE

    def dt(a, b):
        return lax.dot_general(
            a, b, (((1,), (1,)), ((), ())),
            preferred_element_type=jnp.float32,
        ) * inv_t

    logits = jnp.concatenate(
        [dt(n1, p2), dt(p2, n1), dt(n2, p1), dt(p1, n2)], axis=0
    )
    m = jnp.max(logits, axis=1, keepdims=True)
    lse = m + jnp.log(jnp.sum(jnp.exp(logits - m), axis=1, keepdims=True))
    rows = lax.broadcasted_iota(jnp.int32, (4 * _B, _B), 0)
    cols = lax.broadcasted_iota(jnp.int32, (4 * _B, _B), 1)
    picked = jnp.sum(
        jnp.where(cols == lax.rem(rows, _B), logits, 0.0),
        axis=1, keepdims=True,
    )
    out_ref[...] = lse - picked


def _loss(pn, nn):
    return pl.pallas_call(
        _loss_body,
        out_shape=jax.ShapeDtypeStruct((4 * _B, 1), jnp.float32),
    )(pn, nn)


def kernel(projections_1, projections_2, feature_queue):
    P = jnp.concatenate([projections_1, projections_2], axis=0)
    idx2, pn = _simarg(P, feature_queue)
    nn = _sc_gather(feature_queue, idx2.reshape(_B2))
    return _loss(pn, nn).reshape(4 * _B)


# speedup vs baseline: 1.0559x; 1.0559x over previous
import functools

import jax
import jax.numpy as jnp
from jax import lax
from jax.experimental import pallas as pl
from jax.experimental.pallas import tpu as pltpu
from jax.experimental.pallas import tpu_sc as plsc

_TEMPERATURE = 0.1
_B = 128
_B2 = 2 * _B
_D = 64
_Q = 1000000
_BQ = 10000
_NBLK = _Q // _BQ
_NEG = -3.0e38
_BIGF = 3.0e38


def _simarg_body(p_ref, qa_ref, qb_ref, idx_out, pn_out, vmax, vidx, viota):
    i = pl.program_id(0)

    @pl.when(i == 0)
    def _init():
        vmax[...] = jnp.full((1, _B2), _NEG, jnp.float32)
        vidx[...] = jnp.zeros((1, _B2), jnp.int32)
        viota[...] = lax.broadcasted_iota(
            jnp.int32, (_BQ // 2, _B2), 0
        ).astype(jnp.float32)

    p = p_ref[...]
    sq = jnp.sum(p * p, axis=1, keepdims=True)
    pn = p * lax.rsqrt(jnp.maximum(sq, 1e-12))

    for k, q_ref in enumerate((qa_ref, qb_ref)):
        sim = lax.dot_general(
            q_ref[...], pn, (((1,), (1,)), ((), ())),
            preferred_element_type=jnp.float32,
        )
        bmax = jnp.max(sim, axis=0, keepdims=True)
        bloc = jnp.min(
            jnp.where(sim == bmax, viota[...], _BIGF), axis=0, keepdims=True
        )
        bidx = bloc.astype(jnp.int32) + (2 * i + k) * (_BQ // 2)
        better = bmax > vmax[...]
        vmax[...] = jnp.where(better, bmax, vmax[...])
        vidx[...] = jnp.where(better, bidx, vidx[...])

    @pl.when(i == _NBLK - 1)
    def _fin():
        idx_out[...] = vidx[...]
        pn_out[...] = pn


def _simarg(P, Qf):
    return pl.pallas_call(
        _simarg_body,
        grid=(_NBLK,),
        in_specs=[
            pl.BlockSpec((_B2, _D), lambda i: (0, 0)),
            pl.BlockSpec((_BQ // 2, _D), lambda i: (2 * i, 0)),
            pl.BlockSpec((_BQ // 2, _D), lambda i: (2 * i + 1, 0)),
        ],
        out_specs=[
            pl.BlockSpec((1, _B2), lambda i: (0, 0)),
            pl.BlockSpec((_B2, _D), lambda i: (0, 0)),
        ],
        out_shape=[
            jax.ShapeDtypeStruct((1, _B2), jnp.int32),
            jax.ShapeDtypeStruct((_B2, _D), jnp.float32),
        ],
        scratch_shapes=[
            pltpu.VMEM((1, _B2), jnp.float32),
            pltpu.VMEM((1, _B2), jnp.int32),
            pltpu.VMEM((_BQ // 2, _B2), jnp.float32),
        ],
    )(P, Qf, Qf)


def _sc_gather(Q, idx):
    info = plsc.get_sparse_core_info()
    nw = info.num_cores * info.num_subcores
    bpw = _B2 // nw

    mesh = plsc.VectorSubcoreMesh(core_axis_name="c", subcore_axis_name="s")

    @functools.partial(
        pl.kernel,
        mesh=mesh,
        compiler_params=pltpu.CompilerParams(use_tc_tiling_on_sc=False),
        out_type=jax.ShapeDtypeStruct((_B2, _D), jnp.float32),
        scratch_types=[
            pltpu.VMEM((bpw,), jnp.int32),
            pltpu.VMEM((bpw, _D), jnp.float32),
            pltpu.SemaphoreType.DMA,
        ],
    )
    def gk(q_hbm, idx_hbm, out_hbm, idx_v, rows_v, sem):
        wid = lax.axis_index("s") * info.num_cores + lax.axis_index("c")
        base = wid * bpw
        pltpu.sync_copy(idx_hbm.at[pl.ds(base, bpw)], idx_v)
        pltpu.async_copy(q_hbm.at[idx_v], rows_v, sem).wait()
        pltpu.sync_copy(rows_v, out_hbm.at[pl.ds(base, bpw)])

    return gk(Q, idx)


def _loss_body(pn_ref, nn_ref, out_ref):
    pn = pn_ref[...]
    nn = nn_ref[...]
    p1 = pn[:_B]
    p2 = pn[_B:]
    n1 = nn[:_B]
    n2 = nn[_B:]
    inv_t = 1.0 / _TEMPERATURE

    def dt(a, b):
        return lax.dot_general(
            a, b, (((1,), (1,)), ((), ())),
            preferred_element_type=jnp.float32,
        ) * inv_t

    logits = jnp.concatenate(
        [dt(n1, p2), dt(p2, n1), dt(n2, p1), dt(p1, n2)], axis=0
    )
    m = jnp.max(logits, axis=1, keepdims=True)
    lse = m + jnp.log(jnp.sum(jnp.exp(logits - m), axis=1, keepdims=True))
    rows = lax.broadcasted_iota(jnp.int32, (4 * _B, _B), 0)
    cols = lax.broadcasted_iota(jnp.int32, (4 * _B, _B), 1)
    picked = jnp.sum(
        jnp.where(cols == lax.rem(rows, _B), logits, 0.0),
        axis=1, keepdims=True,
    )
    out_ref[...] = lse - picked


def _loss(pn, nn):
    return pl.pallas_call(
        _loss_body,
        out_shape=jax.ShapeDtypeStruct((4 * _B, 1), jnp.float32),
    )(pn, nn)


def kernel(projections_1, projections_2, feature_queue):
    P = jnp.concatenate([projections_1, projections_2], axis=0)
    idx2, pn = _simarg(P, feature_queue)
    nn = _sc_gather(feature_queue, idx2.reshape(_B2))
    return _loss(pn, nn).reshape(4 * _B)


# speedup vs baseline: 1.0706x; 1.0140x over previous
import functools

import jax
import jax.numpy as jnp
from jax import lax
from jax.experimental import pallas as pl
from jax.experimental.pallas import tpu as pltpu
from jax.experimental.pallas import tpu_sc as plsc

_TEMPERATURE = 0.1
_B = 128
_B2 = 2 * _B
_D = 64
_Q = 1000000
_BQ = 10000
_NBLK = _Q // _BQ
_NEG = -3.0e38
_BIGF = 3.0e38


_NS = 5
_BS = _BQ // _NS


def _simarg_body(p_ref, *refs):
    q_refs = refs[:_NS]
    idx_out, pn_out, vmax, vidx, viota = refs[_NS:]
    i = pl.program_id(0)

    @pl.when(i == 0)
    def _init():
        vmax[...] = jnp.full((1, _B2), _NEG, jnp.float32)
        vidx[...] = jnp.zeros((1, _B2), jnp.int32)
        viota[...] = lax.broadcasted_iota(
            jnp.int32, (_BS, _B2), 0
        ).astype(jnp.float32)

    p = p_ref[...]
    sq = jnp.sum(p * p, axis=1, keepdims=True)
    pn = p * lax.rsqrt(jnp.maximum(sq, 1e-12))

    for k, q_ref in enumerate(q_refs):
        sim = lax.dot_general(
            q_ref[...], pn, (((1,), (1,)), ((), ())),
            preferred_element_type=jnp.float32,
        )
        bmax = jnp.max(sim, axis=0, keepdims=True)
        bloc = jnp.min(
            jnp.where(sim == bmax, viota[...], _BIGF), axis=0, keepdims=True
        )
        bidx = bloc.astype(jnp.int32) + (_NS * i + k) * _BS
        better = bmax > vmax[...]
        vmax[...] = jnp.where(better, bmax, vmax[...])
        vidx[...] = jnp.where(better, bidx, vidx[...])

    @pl.when(i == _NBLK - 1)
    def _fin():
        idx_out[...] = vidx[...]
        pn_out[...] = pn


def _simarg(P, Qf):
    def _qspec(k):
        return pl.BlockSpec(
            (_BS, _D), lambda i, _k=k: (_NS * i + _k, 0)
        )

    return pl.pallas_call(
        _simarg_body,
        grid=(_NBLK,),
        in_specs=[pl.BlockSpec((_B2, _D), lambda i: (0, 0))]
        + [_qspec(k) for k in range(_NS)],
        out_specs=[
            pl.BlockSpec((1, _B2), lambda i: (0, 0)),
            pl.BlockSpec((_B2, _D), lambda i: (0, 0)),
        ],
        out_shape=[
            jax.ShapeDtypeStruct((1, _B2), jnp.int32),
            jax.ShapeDtypeStruct((_B2, _D), jnp.float32),
        ],
        scratch_shapes=[
            pltpu.VMEM((1, _B2), jnp.float32),
            pltpu.VMEM((1, _B2), jnp.int32),
            pltpu.VMEM((_BS, _B2), jnp.float32),
        ],
    )(P, *([Qf] * _NS))


def _sc_gather(Q, idx):
    info = plsc.get_sparse_core_info()
    nw = info.num_cores * info.num_subcores
    bpw = _B2 // nw

    mesh = plsc.VectorSubcoreMesh(core_axis_name="c", subcore_axis_name="s")

    @functools.partial(
        pl.kernel,
        mesh=mesh,
        compiler_params=pltpu.CompilerParams(use_tc_tiling_on_sc=False),
        out_type=jax.ShapeDtypeStruct((_B2, _D), jnp.float32),
        scratch_types=[
            pltpu.VMEM((bpw,), jnp.int32),
            pltpu.VMEM((bpw, _D), jnp.float32),
            pltpu.SemaphoreType.DMA,
        ],
    )
    def gk(q_hbm, idx_hbm, out_hbm, idx_v, rows_v, sem):
        wid = lax.axis_index("s") * info.num_cores + lax.axis_index("c")
        base = wid * bpw
        pltpu.sync_copy(idx_hbm.at[pl.ds(base, bpw)], idx_v)
        pltpu.async_copy(q_hbm.at[idx_v], rows_v, sem).wait()
        pltpu.sync_copy(rows_v, out_hbm.at[pl.ds(base, bpw)])

    return gk(Q, idx)


def _loss_body(pn_ref, nn_ref, out_ref):
    pn = pn_ref[...]
    nn = nn_ref[...]
    p1 = pn[:_B]
    p2 = pn[_B:]
    n1 = nn[:_B]
    n2 = nn[_B:]
    inv_t = 1.0 / _TEMPERATURE

    def dt(a, b):
        return lax.dot_general(
            a, b, (((1,), (1,)), ((), ())),
            preferred_element_type=jnp.float32,
        ) * inv_t

    logits = jnp.concatenate(
        [dt(n1, p2), dt(p2, n1), dt(n2, p1), dt(p1, n2)], axis=0
    )
    m = jnp.max(logits, axis=1, keepdims=True)
    lse = m + jnp.log(jnp.sum(jnp.exp(logits - m), axis=1, keepdims=True))
    rows = lax.broadcasted_iota(jnp.int32, (4 * _B, _B), 0)
    cols = lax.broadcasted_iota(jnp.int32, (4 * _B, _B), 1)
    picked = jnp.sum(
        jnp.where(cols == lax.rem(rows, _B), logits, 0.0),
        axis=1, keepdims=True,
    )
    out_ref[...] = lse - picked


def _loss(pn, nn):
    return pl.pallas_call(
        _loss_body,
        out_shape=jax.ShapeDtypeStruct((4 * _B, 1), jnp.float32),
    )(pn, nn)


def kernel(projections_1, projections_2, feature_queue):
    P = jnp.concatenate([projections_1, projections_2], axis=0)
    idx2, pn = _simarg(P, feature_queue)
    nn = _sc_gather(feature_queue, idx2.reshape(_B2))
    return _loss(pn, nn).reshape(4 * _B)
